# Initial kernel scaffold; baseline (speedup 1.0000x reference)
import jax
import jax.numpy as jnp
from jax.experimental import pallas as pl


def kernel(x, edge_index):
    raise NotImplementedError("write your pallas kernel here")


# speedup vs baseline: 5.2671x; 5.2671x over previous
import functools

import jax
import jax.numpy as jnp
from jax import lax
from jax.experimental import pallas as pl
from jax.experimental.pallas import tpu as pltpu
from jax.experimental.pallas import tpu_sc as plsc

N = 10000
E = 320000
D = 128
NC = 2
NS = 16
NW = NC * NS
EW = E // NW
CH = 80
NP = 10240
RT = NP // NS

_QMIN, _QMAX = -128.0, 127.0


def _mesh():
    return plsc.VectorSubcoreMesh(
        core_axis_name="c", subcore_axis_name="s", num_cores=NC,
        num_subcores=NS)


def _rowminmax_body(x_ref, mn_ref, mx_ref):
    mn_ref[...] = jnp.min(x_ref[...], axis=1, keepdims=True)
    mx_ref[...] = jnp.max(x_ref[...], axis=1, keepdims=True)


def _rowminmax(x):
    return pl.pallas_call(
        _rowminmax_body,
        out_shape=(
            jax.ShapeDtypeStruct((N, 1), jnp.float32),
            jax.ShapeDtypeStruct((N, 1), jnp.float32),
        ),
    )(x)


def _msg_minmax_body(rmin_hbm, rmax_hbm, src_hbm, omin_hbm, omax_hbm,
                     rmin_v, rmax_v, idx_v, tmn_v, tmx_v):
    c = lax.axis_index("c")
    s = lax.axis_index("s")
    wid = s * NC + c
    pltpu.sync_copy(rmin_hbm, rmin_v)
    pltpu.sync_copy(rmax_hbm, rmax_v)
    pltpu.sync_copy(src_hbm.at[pl.ds(wid * EW, EW)], idx_v)

    def body(i, carry):
        amn, amx = carry
        idx = idx_v[pl.ds(i * 16, 16)]
        vmn = plsc.load_gather(rmin_v, [idx])
        vmx = plsc.load_gather(rmax_v, [idx])
        return jnp.minimum(amn, vmn), jnp.maximum(amx, vmx)

    init = (jnp.full((16,), jnp.inf, jnp.float32),
            jnp.full((16,), -jnp.inf, jnp.float32))
    amn, amx = lax.fori_loop(0, EW // 16, body, init)
    tmn_v[...] = amn
    tmx_v[...] = amx
    pltpu.sync_copy(tmn_v, omin_hbm.at[wid])
    pltpu.sync_copy(tmx_v, omax_hbm.at[wid])


def _msg_minmax(rmin, rmax, src):
    return pl.kernel(
        _msg_minmax_body,
        out_type=(
            jax.ShapeDtypeStruct((NW, 16), jnp.float32),
            jax.ShapeDtypeStruct((NW, 16), jnp.float32),
        ),
        mesh=_mesh(),
        compiler_params=pltpu.CompilerParams(needs_layout_passes=False),
        scratch_types=[
            pltpu.VMEM((N,), jnp.float32),
            pltpu.VMEM((N,), jnp.float32),
            pltpu.VMEM((EW,), jnp.int32),
            pltpu.VMEM((16,), jnp.float32),
            pltpu.VMEM((16,), jnp.float32),
        ],
    )(rmin, rmax, src)


def _quant_x_body(x_ref, omin_ref, omax_ref, qx_ref):
    mn = jnp.minimum(jnp.min(omin_ref[...]), 0.0)
    mx = jnp.maximum(jnp.max(omax_ref[...]), 0.0)
    scale = jnp.maximum((mx - mn) / (_QMAX - _QMIN), 1e-8)
    zp = _QMIN - jnp.round(mn / scale)
    q = jnp.clip(jnp.round(x_ref[...] / scale) + zp, _QMIN, _QMAX)
    qx_ref[...] = (q - zp) * scale


def _quant_x(x, omin, omax):
    return pl.pallas_call(
        _quant_x_body,
        out_shape=jax.ShapeDtypeStruct((N, D), jnp.float32),
    )(x, omin, omax)


def _scatter_body(qx_hbm, src_hbm, dst_hbm, zer_hbm, out_hbm,
                  acc, srcb, dstb, rows, sem):
    c = lax.axis_index("c")
    s = lax.axis_index("s")
    pltpu.sync_copy(zer_hbm, acc.at[pl.ds(s * RT, RT)])
    plsc.subcore_barrier()
    ebase = c * (E // NC) + s * EW

    def body(k, carry):
        off = ebase + k * CH
        pltpu.sync_copy(src_hbm.at[pl.ds(off, CH)], srcb)
        pltpu.sync_copy(dst_hbm.at[pl.ds(off, CH)], dstb)
        pltpu.async_copy(qx_hbm.at[srcb], rows, sem).wait()
        pltpu.sync_copy(rows, acc.at[dstb], add=True)
        return carry

    lax.fori_loop(0, EW // CH, body, 0)
    plsc.subcore_barrier()
    pltpu.sync_copy(acc.at[pl.ds(s * RT, RT)],
                    out_hbm.at[pl.ds(c * NP + s * RT, RT)])


def _scatter_agg(qx, src, dst, zer):
    return pl.kernel(
        _scatter_body,
        out_type=jax.ShapeDtypeStruct((NC * NP, D), jnp.float32),
        mesh=_mesh(),
        scratch_types=[
            pltpu.VMEM_SHARED((NP, D), jnp.float32),
            pltpu.VMEM((CH,), jnp.int32),
            pltpu.VMEM((CH,), jnp.int32),
            pltpu.VMEM((CH, D), jnp.float32),
            pltpu.SemaphoreType.DMA,
        ],
    )(qx, src, dst, zer)


_BLK = 2048
_NBLK = NP // _BLK


def _sum_minmax_body(a0_ref, a1_ref, s_ref, mn_ref, mx_ref, smn, smx):
    i = pl.program_id(0)
    t = a0_ref[...] + a1_ref[...]
    s_ref[...] = t
    bmn = jnp.min(t)
    bmx = jnp.max(t)

    @pl.when(i == 0)
    def _():
        smn[0] = bmn
        smx[0] = bmx

    @pl.when(i > 0)
    def _():
        smn[0] = jnp.minimum(smn[0], bmn)
        smx[0] = jnp.maximum(smx[0], bmx)

    @pl.when(i == _NBLK - 1)
    def _():
        mn_ref[0, 0] = smn[0]
        mx_ref[0, 0] = smx[0]


def _sum_minmax(agg2):
    return pl.pallas_call(
        _sum_minmax_body,
        grid=(_NBLK,),
        in_specs=[
            pl.BlockSpec((_BLK, D), lambda i: (i, 0)),
            pl.BlockSpec((_BLK, D), lambda i: (i + _NBLK, 0)),
        ],
        out_specs=(
            pl.BlockSpec((_BLK, D), lambda i: (i, 0)),
            pl.BlockSpec(memory_space=pltpu.SMEM),
            pl.BlockSpec(memory_space=pltpu.SMEM),
        ),
        out_shape=(
            jax.ShapeDtypeStruct((NP, D), jnp.float32),
            jax.ShapeDtypeStruct((1, 1), jnp.float32),
            jax.ShapeDtypeStruct((1, 1), jnp.float32),
        ),
        scratch_shapes=[
            pltpu.SMEM((1,), jnp.float32),
            pltpu.SMEM((1,), jnp.float32),
        ],
    )(agg2, agg2)


def _qparams(mn, mx):
    scale = jnp.maximum((mx - mn) / (_QMAX - _QMIN), 1e-8)
    zp = _QMIN - jnp.round(mn / scale)
    return scale, zp


def _fq(v, scale, zp):
    q = jnp.clip(jnp.round(v / scale) + zp, _QMIN, _QMAX)
    return (q - zp) * scale


def _double_quant_body(s_ref, mn_ref, mx_ref, o_ref):
    mn_s = mn_ref[0, 0]
    mx_s = mx_ref[0, 0]
    mn2 = jnp.minimum(mn_s, 0.0)
    mx2 = jnp.maximum(mx_s, 0.0)
    sc2, zp2 = _qparams(mn2, mx2)
    dq2 = _fq(s_ref[...], sc2, zp2)
    mn3 = jnp.minimum(_fq(mn_s, sc2, zp2), 0.0)
    mx3 = jnp.maximum(_fq(mx_s, sc2, zp2), 0.0)
    sc3, zp3 = _qparams(mn3, mx3)
    o_ref[...] = _fq(dq2, sc3, zp3)


def _double_quant(ssum, mn, mx):
    return pl.pallas_call(
        _double_quant_body,
        grid=(_NBLK,),
        in_specs=[
            pl.BlockSpec((_BLK, D), lambda i: (i, 0)),
            pl.BlockSpec(memory_space=pltpu.SMEM),
            pl.BlockSpec(memory_space=pltpu.SMEM),
        ],
        out_specs=pl.BlockSpec((_BLK, D), lambda i: (i, 0)),
        out_shape=jax.ShapeDtypeStruct((NP, D), jnp.float32),
    )(ssum, mn, mx)


def kernel(x, edge_index):
    src = edge_index[0].astype(jnp.int32)
    dst = edge_index[1].astype(jnp.int32)
    x = x.astype(jnp.float32)

    rmin, rmax = _rowminmax(x)
    omin, omax = _msg_minmax(rmin.reshape(N), rmax.reshape(N), src)
    qx = _quant_x(x, omin, omax)
    zer = jnp.zeros((RT, D), jnp.float32)
    agg2 = _scatter_agg(qx, src, dst, zer)
    ssum, mn, mx = _sum_minmax(agg2)
    return _double_quant(ssum, mn, mx)[:N]


# speedup vs baseline: 8.5395x; 1.6213x over previous
import functools

import jax
import jax.numpy as jnp
from jax import lax
from jax.experimental import pallas as pl
from jax.experimental.pallas import tpu as pltpu
from jax.experimental.pallas import tpu_sc as plsc

N = 10000
E = 320000
D = 128
NC = 2
NS = 16
NW = NC * NS
EW = E // NW
CH = 80
NP = 10240
RT = NP // NS

_QMIN, _QMAX = -128.0, 127.0


def _mesh():
    return plsc.VectorSubcoreMesh(
        core_axis_name="c", subcore_axis_name="s", num_cores=NC,
        num_subcores=NS)


def _rowminmax_body(x_ref, mn_ref, mx_ref):
    mn_ref[...] = jnp.min(x_ref[...], axis=1, keepdims=True)
    mx_ref[...] = jnp.max(x_ref[...], axis=1, keepdims=True)


def _rowminmax(x):
    return pl.pallas_call(
        _rowminmax_body,
        out_shape=(
            jax.ShapeDtypeStruct((N, 1), jnp.float32),
            jax.ShapeDtypeStruct((N, 1), jnp.float32),
        ),
    )(x)


def _msg_minmax_body(rmin_hbm, rmax_hbm, src_hbm, omin_hbm, omax_hbm,
                     rmin_v, rmax_v, idx_v, tmn_v, tmx_v):
    c = lax.axis_index("c")
    s = lax.axis_index("s")
    wid = s * NC + c
    pltpu.sync_copy(rmin_hbm, rmin_v)
    pltpu.sync_copy(rmax_hbm, rmax_v)
    pltpu.sync_copy(src_hbm.at[pl.ds(wid * EW, EW)], idx_v)

    def body(i, carry):
        amn, amx = carry
        idx = idx_v[pl.ds(i * 16, 16)]
        vmn = plsc.load_gather(rmin_v, [idx])
        vmx = plsc.load_gather(rmax_v, [idx])
        return jnp.minimum(amn, vmn), jnp.maximum(amx, vmx)

    init = (jnp.full((16,), jnp.inf, jnp.float32),
            jnp.full((16,), -jnp.inf, jnp.float32))
    amn, amx = lax.fori_loop(0, EW // 16, body, init)
    tmn_v[...] = amn
    tmx_v[...] = amx
    pltpu.sync_copy(tmn_v, omin_hbm.at[wid])
    pltpu.sync_copy(tmx_v, omax_hbm.at[wid])


def _msg_minmax(rmin, rmax, src):
    return pl.kernel(
        _msg_minmax_body,
        out_type=(
            jax.ShapeDtypeStruct((NW, 16), jnp.float32),
            jax.ShapeDtypeStruct((NW, 16), jnp.float32),
        ),
        mesh=_mesh(),
        compiler_params=pltpu.CompilerParams(needs_layout_passes=False),
        scratch_types=[
            pltpu.VMEM((N,), jnp.float32),
            pltpu.VMEM((N,), jnp.float32),
            pltpu.VMEM((EW,), jnp.int32),
            pltpu.VMEM((16,), jnp.float32),
            pltpu.VMEM((16,), jnp.float32),
        ],
    )(rmin, rmax, src)


def _quant_x_body(x_ref, omin_ref, omax_ref, qx_ref):
    mn = jnp.minimum(jnp.min(omin_ref[...]), 0.0)
    mx = jnp.maximum(jnp.max(omax_ref[...]), 0.0)
    scale = jnp.maximum((mx - mn) / (_QMAX - _QMIN), 1e-8)
    zp = _QMIN - jnp.round(mn / scale)
    q = jnp.clip(jnp.round(x_ref[...] / scale) + zp, _QMIN, _QMAX)
    qx_ref[...] = (q - zp) * scale


def _quant_x(x, omin, omax):
    return pl.pallas_call(
        _quant_x_body,
        out_shape=jax.ShapeDtypeStruct((N, D), jnp.float32),
    )(x, omin, omax)


NCH = EW // CH


def _scatter_body(qx_hbm, src_hbm, dst_hbm, zer_hbm, out_hbm,
                  acc, srcb, dstb, rows, sg, ss, si, di):
    c = lax.axis_index("c")
    s = lax.axis_index("s")
    pltpu.sync_copy(zer_hbm, acc.at[pl.ds(s * RT, RT)])
    plsc.subcore_barrier()
    ebase = c * (E // NC) + s * EW

    def start_idx(k, j):
        off = ebase + k * CH
        pltpu.async_copy(src_hbm.at[pl.ds(off, CH)], srcb[j], si[j])
        pltpu.async_copy(dst_hbm.at[pl.ds(off, CH)], dstb[j], di[j])

    def wait_idx(j):
        pltpu.make_async_copy(src_hbm.at[pl.ds(0, CH)], srcb[j], si[j]).wait()
        pltpu.make_async_copy(dst_hbm.at[pl.ds(0, CH)], dstb[j], di[j]).wait()

    def gather(b, j):
        pltpu.async_copy(qx_hbm.at[srcb[j]], rows[b], sg[b]).wait()

    def start_scatter(b, j):
        pltpu.async_copy(rows[b], acc.at[dstb[j]], ss[b], add=True)

    def wait_scatter(b):
        pltpu.make_async_copy(qx_hbm.at[pl.ds(0, CH)], rows[b], ss[b]).wait()

    def step(k, kk):
        b, j = kk % 2, kk % 3
        wait_scatter(b)
        start_idx(k + 1, (j + 1) % 3)
        wait_idx(j)
        gather(b, j)
        start_scatter(b, j)

    start_idx(0, 0)
    start_idx(1, 1)
    wait_idx(0)
    gather(0, 0)
    start_idx(2, 2)
    start_scatter(0, 0)
    wait_idx(1)
    gather(1, 1)
    start_scatter(1, 1)

    def body(g, carry):
        k0 = 2 + g * 6
        for i in range(6):
            step(k0 + i, 2 + i)
        return carry

    lax.fori_loop(0, (NCH - 5) // 6, body, 0)

    for k in (122, 123, 124):
        b, j = k % 2, k % 3
        wait_scatter(b)
        if k + 1 < NCH:
            start_idx(k + 1, (j + 1) % 3)
        wait_idx(j)
        gather(b, j)
        start_scatter(b, j)
    wait_scatter(NCH % 2)
    wait_scatter((NCH + 1) % 2)

    plsc.subcore_barrier()
    pltpu.sync_copy(acc.at[pl.ds(s * RT, RT)],
                    out_hbm.at[pl.ds(c * NP + s * RT, RT)])


def _scatter_agg(qx, src, dst, zer):
    return pl.kernel(
        _scatter_body,
        out_type=jax.ShapeDtypeStruct((NC * NP, D), jnp.float32),
        mesh=_mesh(),
        scratch_types=[
            pltpu.VMEM_SHARED((NP, D), jnp.float32),
            [pltpu.VMEM((CH,), jnp.int32) for _ in range(3)],
            [pltpu.VMEM((CH,), jnp.int32) for _ in range(3)],
            [pltpu.VMEM((CH, D), jnp.float32) for _ in range(2)],
            [pltpu.SemaphoreType.DMA for _ in range(2)],
            [pltpu.SemaphoreType.DMA for _ in range(2)],
            [pltpu.SemaphoreType.DMA for _ in range(3)],
            [pltpu.SemaphoreType.DMA for _ in range(3)],
        ],
    )(qx, src, dst, zer)


_BLK = 2048
_NBLK = NP // _BLK


def _sum_minmax_body(a0_ref, a1_ref, s_ref, mn_ref, mx_ref, smn, smx):
    i = pl.program_id(0)
    t = a0_ref[...] + a1_ref[...]
    s_ref[...] = t
    bmn = jnp.min(t)
    bmx = jnp.max(t)

    @pl.when(i == 0)
    def _():
        smn[0] = bmn
        smx[0] = bmx

    @pl.when(i > 0)
    def _():
        smn[0] = jnp.minimum(smn[0], bmn)
        smx[0] = jnp.maximum(smx[0], bmx)

    @pl.when(i == _NBLK - 1)
    def _():
        mn_ref[0, 0] = smn[0]
        mx_ref[0, 0] = smx[0]


def _sum_minmax(agg2):
    return pl.pallas_call(
        _sum_minmax_body,
        grid=(_NBLK,),
        in_specs=[
            pl.BlockSpec((_BLK, D), lambda i: (i, 0)),
            pl.BlockSpec((_BLK, D), lambda i: (i + _NBLK, 0)),
        ],
        out_specs=(
            pl.BlockSpec((_BLK, D), lambda i: (i, 0)),
            pl.BlockSpec(memory_space=pltpu.SMEM),
            pl.BlockSpec(memory_space=pltpu.SMEM),
        ),
        out_shape=(
            jax.ShapeDtypeStruct((NP, D), jnp.float32),
            jax.ShapeDtypeStruct((1, 1), jnp.float32),
            jax.ShapeDtypeStruct((1, 1), jnp.float32),
        ),
        scratch_shapes=[
            pltpu.SMEM((1,), jnp.float32),
            pltpu.SMEM((1,), jnp.float32),
        ],
    )(agg2, agg2)


def _qparams(mn, mx):
    scale = jnp.maximum((mx - mn) / (_QMAX - _QMIN), 1e-8)
    zp = _QMIN - jnp.round(mn / scale)
    return scale, zp


def _fq(v, scale, zp):
    q = jnp.clip(jnp.round(v / scale) + zp, _QMIN, _QMAX)
    return (q - zp) * scale


def _double_quant_body(s_ref, mn_ref, mx_ref, o_ref):
    mn_s = mn_ref[0, 0]
    mx_s = mx_ref[0, 0]
    mn2 = jnp.minimum(mn_s, 0.0)
    mx2 = jnp.maximum(mx_s, 0.0)
    sc2, zp2 = _qparams(mn2, mx2)
    dq2 = _fq(s_ref[...], sc2, zp2)
    mn3 = jnp.minimum(_fq(mn_s, sc2, zp2), 0.0)
    mx3 = jnp.maximum(_fq(mx_s, sc2, zp2), 0.0)
    sc3, zp3 = _qparams(mn3, mx3)
    o_ref[...] = _fq(dq2, sc3, zp3)


def _double_quant(ssum, mn, mx):
    return pl.pallas_call(
        _double_quant_body,
        grid=(_NBLK,),
        in_specs=[
            pl.BlockSpec((_BLK, D), lambda i: (i, 0)),
            pl.BlockSpec(memory_space=pltpu.SMEM),
            pl.BlockSpec(memory_space=pltpu.SMEM),
        ],
        out_specs=pl.BlockSpec((_BLK, D), lambda i: (i, 0)),
        out_shape=jax.ShapeDtypeStruct((NP, D), jnp.float32),
    )(ssum, mn, mx)


def kernel(x, edge_index):
    src = edge_index[0].astype(jnp.int32)
    dst = edge_index[1].astype(jnp.int32)
    x = x.astype(jnp.float32)

    rmin, rmax = _rowminmax(x)
    omin, omax = _msg_minmax(rmin.reshape(N), rmax.reshape(N), src)
    qx = _quant_x(x, omin, omax)
    zer = jnp.zeros((RT, D), jnp.float32)
    agg2 = _scatter_agg(qx, src, dst, zer)
    ssum, mn, mx = _sum_minmax(agg2)
    return _double_quant(ssum, mn, mx)[:N]


# speedup vs baseline: 9.5659x; 1.1202x over previous
import functools

import jax
import jax.numpy as jnp
from jax import lax
from jax.experimental import pallas as pl
from jax.experimental.pallas import tpu as pltpu
from jax.experimental.pallas import tpu_sc as plsc

N = 10000
E = 320000
D = 128
NC = 2
NS = 16
NW = NC * NS
EW = E // NW
CH = 128
CT = 16
NP = 10240
RT = NP // NS

_QMIN, _QMAX = -128.0, 127.0


def _mesh():
    return plsc.VectorSubcoreMesh(
        core_axis_name="c", subcore_axis_name="s", num_cores=NC,
        num_subcores=NS)


def _rowminmax_body(x_ref, mn_ref, mx_ref):
    mn_ref[...] = jnp.min(x_ref[...], axis=1, keepdims=True)
    mx_ref[...] = jnp.max(x_ref[...], axis=1, keepdims=True)


def _rowminmax(x):
    return pl.pallas_call(
        _rowminmax_body,
        out_shape=(
            jax.ShapeDtypeStruct((N, 1), jnp.float32),
            jax.ShapeDtypeStruct((N, 1), jnp.float32),
        ),
    )(x)


def _msg_minmax_body(rmin_hbm, rmax_hbm, src_hbm, omin_hbm, omax_hbm,
                     rmin_v, rmax_v, idx_v, tmn_v, tmx_v):
    c = lax.axis_index("c")
    s = lax.axis_index("s")
    wid = s * NC + c
    pltpu.sync_copy(rmin_hbm, rmin_v)
    pltpu.sync_copy(rmax_hbm, rmax_v)
    pltpu.sync_copy(src_hbm.at[pl.ds(wid * EW, EW)], idx_v)

    def body(i, carry):
        amn, amx = carry
        idx = idx_v[pl.ds(i * 16, 16)]
        vmn = plsc.load_gather(rmin_v, [idx])
        vmx = plsc.load_gather(rmax_v, [idx])
        return jnp.minimum(amn, vmn), jnp.maximum(amx, vmx)

    init = (jnp.full((16,), jnp.inf, jnp.float32),
            jnp.full((16,), -jnp.inf, jnp.float32))
    amn, amx = lax.fori_loop(0, EW // 16, body, init)
    tmn_v[...] = amn
    tmx_v[...] = amx
    pltpu.sync_copy(tmn_v, omin_hbm.at[wid])
    pltpu.sync_copy(tmx_v, omax_hbm.at[wid])


def _msg_minmax(rmin, rmax, src):
    return pl.kernel(
        _msg_minmax_body,
        out_type=(
            jax.ShapeDtypeStruct((NW, 16), jnp.float32),
            jax.ShapeDtypeStruct((NW, 16), jnp.float32),
        ),
        mesh=_mesh(),
        compiler_params=pltpu.CompilerParams(needs_layout_passes=False),
        scratch_types=[
            pltpu.VMEM((N,), jnp.float32),
            pltpu.VMEM((N,), jnp.float32),
            pltpu.VMEM((EW,), jnp.int32),
            pltpu.VMEM((16,), jnp.float32),
            pltpu.VMEM((16,), jnp.float32),
        ],
    )(rmin, rmax, src)


def _quant_x_body(x_ref, omin_ref, omax_ref, qx_ref):
    mn = jnp.minimum(jnp.min(omin_ref[...]), 0.0)
    mx = jnp.maximum(jnp.max(omax_ref[...]), 0.0)
    scale = jnp.maximum((mx - mn) / (_QMAX - _QMIN), 1e-8)
    zp = _QMIN - jnp.round(mn / scale)
    q = jnp.clip(jnp.round(x_ref[...] / scale) + zp, _QMIN, _QMAX)
    qx_ref[...] = (q - zp) * scale


def _quant_x(x, omin, omax):
    return pl.pallas_call(
        _quant_x_body,
        out_shape=jax.ShapeDtypeStruct((N, D), jnp.float32),
    )(x, omin, omax)


NFULL = EW // CH


def _scatter_body(qx_hbm, src_hbm, dst_hbm, zer_hbm, out_hbm,
                  acc, srcb, dstb, rows, srct, dstt, rowst,
                  sg, ss, si, di, st):
    c = lax.axis_index("c")
    s = lax.axis_index("s")
    pltpu.sync_copy(zer_hbm, acc.at[pl.ds(s * RT, RT)])
    plsc.subcore_barrier()
    ebase = c * (E // NC) + s * EW

    def start_idx(k, j):
        off = ebase + k * CH
        pltpu.async_copy(src_hbm.at[pl.ds(off, CH)], srcb[j], si[j])
        pltpu.async_copy(dst_hbm.at[pl.ds(off, CH)], dstb[j], di[j])

    def wait_idx(j):
        pltpu.make_async_copy(src_hbm.at[pl.ds(0, CH)], srcb[j], si[j]).wait()
        pltpu.make_async_copy(dst_hbm.at[pl.ds(0, CH)], dstb[j], di[j]).wait()

    def gather(b, j):
        pltpu.async_copy(qx_hbm.at[srcb[j]], rows[b], sg[b]).wait()

    def start_scatter(b, j):
        pltpu.async_copy(rows[b], acc.at[dstb[j]], ss[b], add=True)

    def wait_scatter(b):
        pltpu.make_async_copy(qx_hbm.at[pl.ds(0, CH)], rows[b], ss[b]).wait()

    def step(k, kk, prefetch=True):
        b, j = kk % 2, kk % 3
        wait_scatter(b)
        if prefetch:
            start_idx(k + 1, (j + 1) % 3)
        wait_idx(j)
        gather(b, j)
        start_scatter(b, j)

    start_idx(0, 0)
    start_idx(1, 1)
    wait_idx(0)
    gather(0, 0)
    start_idx(2, 2)
    start_scatter(0, 0)
    wait_idx(1)
    gather(1, 1)
    start_scatter(1, 1)

    def body(g, carry):
        k0 = 2 + g * 6
        for i in range(6):
            step(k0 + i, 2 + i)
        return carry

    lax.fori_loop(0, 12, body, 0)

    for k in (74, 75, 76, 77):
        step(k, k, prefetch=(k + 1 < NFULL))
    wait_scatter(NFULL % 2)
    wait_scatter((NFULL + 1) % 2)

    toff = ebase + NFULL * CH
    pltpu.sync_copy(src_hbm.at[pl.ds(toff, CT)], srct)
    pltpu.sync_copy(dst_hbm.at[pl.ds(toff, CT)], dstt)
    pltpu.async_copy(qx_hbm.at[srct], rowst, st).wait()
    pltpu.sync_copy(rowst, acc.at[dstt], add=True)

    plsc.subcore_barrier()
    pltpu.sync_copy(acc.at[pl.ds(s * RT, RT)],
                    out_hbm.at[pl.ds(c * NP + s * RT, RT)])


def _scatter_agg(qx, src, dst, zer):
    return pl.kernel(
        _scatter_body,
        out_type=jax.ShapeDtypeStruct((NC * NP, D), jnp.float32),
        mesh=_mesh(),
        scratch_types=[
            pltpu.VMEM_SHARED((NP, D), jnp.float32),
            [pltpu.VMEM((CH,), jnp.int32) for _ in range(3)],
            [pltpu.VMEM((CH,), jnp.int32) for _ in range(3)],
            [pltpu.VMEM((CH, D), jnp.float32) for _ in range(2)],
            pltpu.VMEM((CT,), jnp.int32),
            pltpu.VMEM((CT,), jnp.int32),
            pltpu.VMEM((CT, D), jnp.float32),
            [pltpu.SemaphoreType.DMA for _ in range(2)],
            [pltpu.SemaphoreType.DMA for _ in range(2)],
            [pltpu.SemaphoreType.DMA for _ in range(3)],
            [pltpu.SemaphoreType.DMA for _ in range(3)],
            pltpu.SemaphoreType.DMA,
        ],
    )(qx, src, dst, zer)


_BLK = 2048
_NBLK = NP // _BLK


def _sum_minmax_body(a0_ref, a1_ref, s_ref, mn_ref, mx_ref, smn, smx):
    i = pl.program_id(0)
    t = a0_ref[...] + a1_ref[...]
    s_ref[...] = t
    bmn = jnp.min(t)
    bmx = jnp.max(t)

    @pl.when(i == 0)
    def _():
        smn[0] = bmn
        smx[0] = bmx

    @pl.when(i > 0)
    def _():
        smn[0] = jnp.minimum(smn[0], bmn)
        smx[0] = jnp.maximum(smx[0], bmx)

    @pl.when(i == _NBLK - 1)
    def _():
        mn_ref[0, 0] = smn[0]
        mx_ref[0, 0] = smx[0]


def _sum_minmax(agg2):
    return pl.pallas_call(
        _sum_minmax_body,
        grid=(_NBLK,),
        in_specs=[
            pl.BlockSpec((_BLK, D), lambda i: (i, 0)),
            pl.BlockSpec((_BLK, D), lambda i: (i + _NBLK, 0)),
        ],
        out_specs=(
            pl.BlockSpec((_BLK, D), lambda i: (i, 0)),
            pl.BlockSpec(memory_space=pltpu.SMEM),
            pl.BlockSpec(memory_space=pltpu.SMEM),
        ),
        out_shape=(
            jax.ShapeDtypeStruct((NP, D), jnp.float32),
            jax.ShapeDtypeStruct((1, 1), jnp.float32),
            jax.ShapeDtypeStruct((1, 1), jnp.float32),
        ),
        scratch_shapes=[
            pltpu.SMEM((1,), jnp.float32),
            pltpu.SMEM((1,), jnp.float32),
        ],
    )(agg2, agg2)


def _qparams(mn, mx):
    scale = jnp.maximum((mx - mn) / (_QMAX - _QMIN), 1e-8)
    zp = _QMIN - jnp.round(mn / scale)
    return scale, zp


def _fq(v, scale, zp):
    q = jnp.clip(jnp.round(v / scale) + zp, _QMIN, _QMAX)
    return (q - zp) * scale


def _double_quant_body(s_ref, mn_ref, mx_ref, o_ref):
    mn_s = mn_ref[0, 0]
    mx_s = mx_ref[0, 0]
    mn2 = jnp.minimum(mn_s, 0.0)
    mx2 = jnp.maximum(mx_s, 0.0)
    sc2, zp2 = _qparams(mn2, mx2)
    dq2 = _fq(s_ref[...], sc2, zp2)
    mn3 = jnp.minimum(_fq(mn_s, sc2, zp2), 0.0)
    mx3 = jnp.maximum(_fq(mx_s, sc2, zp2), 0.0)
    sc3, zp3 = _qparams(mn3, mx3)
    o_ref[...] = _fq(dq2, sc3, zp3)


def _double_quant(ssum, mn, mx):
    return pl.pallas_call(
        _double_quant_body,
        grid=(_NBLK,),
        in_specs=[
            pl.BlockSpec((_BLK, D), lambda i: (i, 0)),
            pl.BlockSpec(memory_space=pltpu.SMEM),
            pl.BlockSpec(memory_space=pltpu.SMEM),
        ],
        out_specs=pl.BlockSpec((_BLK, D), lambda i: (i, 0)),
        out_shape=jax.ShapeDtypeStruct((NP, D), jnp.float32),
    )(ssum, mn, mx)


def kernel(x, edge_index):
    src = edge_index[0].astype(jnp.int32)
    dst = edge_index[1].astype(jnp.int32)
    x = x.astype(jnp.float32)

    rmin, rmax = _rowminmax(x)
    omin, omax = _msg_minmax(rmin.reshape(N), rmax.reshape(N), src)
    qx = _quant_x(x, omin, omax)
    zer = jnp.zeros((RT, D), jnp.float32)
    agg2 = _scatter_agg(qx, src, dst, zer)
    ssum, mn, mx = _sum_minmax(agg2)
    return _double_quant(ssum, mn, mx)[:N]


# speedup vs baseline: 9.7222x; 1.0163x over previous
import functools

import jax
import jax.numpy as jnp
from jax import lax
from jax.experimental import pallas as pl
from jax.experimental.pallas import tpu as pltpu
from jax.experimental.pallas import tpu_sc as plsc

N = 10000
E = 320000
D = 128
NC = 2
NS = 16
NW = NC * NS
EW = E // NW
CH = 128
CT = 16
NP = 10240
RT = NP // NS

_QMIN, _QMAX = -128.0, 127.0


def _mesh():
    return plsc.VectorSubcoreMesh(
        core_axis_name="c", subcore_axis_name="s", num_cores=NC,
        num_subcores=NS)


def _rowminmax_body(x_ref, mn_ref, mx_ref):
    mn_ref[...] = jnp.min(x_ref[...], axis=1, keepdims=True)
    mx_ref[...] = jnp.max(x_ref[...], axis=1, keepdims=True)


def _rowminmax(x):
    return pl.pallas_call(
        _rowminmax_body,
        out_shape=(
            jax.ShapeDtypeStruct((N, 1), jnp.float32),
            jax.ShapeDtypeStruct((N, 1), jnp.float32),
        ),
    )(x)


def _msg_minmax_body(rmin_hbm, rmax_hbm, src_hbm, omin_hbm, omax_hbm,
                     rmin_v, rmax_v, idx_v, tmn_v, tmx_v):
    c = lax.axis_index("c")
    s = lax.axis_index("s")
    wid = s * NC + c
    pltpu.sync_copy(rmin_hbm, rmin_v)
    pltpu.sync_copy(rmax_hbm, rmax_v)
    pltpu.sync_copy(src_hbm.at[pl.ds(wid * EW, EW)], idx_v)

    def body(i, carry):
        amn, amx = carry
        idx = idx_v[pl.ds(i * 16, 16)]
        vmn = plsc.load_gather(rmin_v, [idx])
        vmx = plsc.load_gather(rmax_v, [idx])
        return jnp.minimum(amn, vmn), jnp.maximum(amx, vmx)

    init = (jnp.full((16,), jnp.inf, jnp.float32),
            jnp.full((16,), -jnp.inf, jnp.float32))
    amn, amx = lax.fori_loop(0, EW // 16, body, init)
    tmn_v[...] = amn
    tmx_v[...] = amx
    pltpu.sync_copy(tmn_v, omin_hbm.at[wid])
    pltpu.sync_copy(tmx_v, omax_hbm.at[wid])


def _msg_minmax(rmin, rmax, src):
    return pl.kernel(
        _msg_minmax_body,
        out_type=(
            jax.ShapeDtypeStruct((NW, 16), jnp.float32),
            jax.ShapeDtypeStruct((NW, 16), jnp.float32),
        ),
        mesh=_mesh(),
        compiler_params=pltpu.CompilerParams(needs_layout_passes=False),
        scratch_types=[
            pltpu.VMEM((N,), jnp.float32),
            pltpu.VMEM((N,), jnp.float32),
            pltpu.VMEM((EW,), jnp.int32),
            pltpu.VMEM((16,), jnp.float32),
            pltpu.VMEM((16,), jnp.float32),
        ],
    )(rmin, rmax, src)


def _quant_x_body(x_ref, omin_ref, omax_ref, qx_ref):
    mn = jnp.minimum(jnp.min(omin_ref[...]), 0.0)
    mx = jnp.maximum(jnp.max(omax_ref[...]), 0.0)
    scale = jnp.maximum((mx - mn) / (_QMAX - _QMIN), 1e-8)
    zp = _QMIN - jnp.round(mn / scale)
    q = jnp.clip(jnp.round(x_ref[...] / scale) + zp, _QMIN, _QMAX)
    qx_ref[...] = (q - zp) * scale


def _quant_x(x, omin, omax):
    return pl.pallas_call(
        _quant_x_body,
        out_shape=jax.ShapeDtypeStruct((N, D), jnp.float32),
    )(x, omin, omax)


NFULL = EW // CH


def _scatter_body(qx_hbm, src_hbm, dst_hbm, zer_hbm, out_hbm,
                  acc, srcb, dstb, rows, srct, dstt, rowst,
                  sg, ss, si, di, st):
    c = lax.axis_index("c")
    s = lax.axis_index("s")
    ebase = c * (E // NC) + s * EW

    def start_idx(k, j):
        off = ebase + k * CH
        pltpu.async_copy(src_hbm.at[pl.ds(off, CH)], srcb[j], si[j])
        pltpu.async_copy(dst_hbm.at[pl.ds(off, CH)], dstb[j], di[j])

    def wait_idx(j):
        pltpu.make_async_copy(src_hbm.at[pl.ds(0, CH)], srcb[j], si[j]).wait()
        pltpu.make_async_copy(dst_hbm.at[pl.ds(0, CH)], dstb[j], di[j]).wait()

    def gather(b, j):
        pltpu.async_copy(qx_hbm.at[srcb[j]], rows[b], sg[b]).wait()

    def start_scatter(b, j):
        pltpu.async_copy(rows[b], acc.at[dstb[j]], ss[b], add=True)

    def wait_scatter(b):
        pltpu.make_async_copy(qx_hbm.at[pl.ds(0, CH)], rows[b], ss[b]).wait()

    def step(k, kk, prefetch=True):
        b, j = kk % 2, kk % 3
        wait_scatter(b)
        if prefetch:
            start_idx(k + 1, (j + 1) % 3)
        wait_idx(j)
        gather(b, j)
        start_scatter(b, j)

    pltpu.async_copy(zer_hbm, acc.at[pl.ds(s * RT, RT)], st)
    start_idx(0, 0)
    start_idx(1, 1)
    wait_idx(0)
    gather(0, 0)
    start_idx(2, 2)
    pltpu.make_async_copy(zer_hbm, acc.at[pl.ds(s * RT, RT)], st).wait()
    plsc.subcore_barrier()
    start_scatter(0, 0)
    wait_idx(1)
    gather(1, 1)
    start_scatter(1, 1)

    def body(g, carry):
        k0 = 2 + g * 6
        for i in range(6):
            step(k0 + i, 2 + i)
        return carry

    lax.fori_loop(0, 12, body, 0)

    for k in (74, 75, 76, 77):
        step(k, k, prefetch=(k + 1 < NFULL))
    wait_scatter(NFULL % 2)
    wait_scatter((NFULL + 1) % 2)

    toff = ebase + NFULL * CH
    pltpu.sync_copy(src_hbm.at[pl.ds(toff, CT)], srct)
    pltpu.sync_copy(dst_hbm.at[pl.ds(toff, CT)], dstt)
    pltpu.async_copy(qx_hbm.at[srct], rowst, st).wait()
    pltpu.sync_copy(rowst, acc.at[dstt], add=True)

    plsc.subcore_barrier()
    pltpu.sync_copy(acc.at[pl.ds(s * RT, RT)],
                    out_hbm.at[pl.ds(c * NP + s * RT, RT)])


def _scatter_agg(qx, src, dst, zer):
    return pl.kernel(
        _scatter_body,
        out_type=jax.ShapeDtypeStruct((NC * NP, D), jnp.float32),
        mesh=_mesh(),
        scratch_types=[
            pltpu.VMEM_SHARED((NP, D), jnp.float32),
            [pltpu.VMEM((CH,), jnp.int32) for _ in range(3)],
            [pltpu.VMEM((CH,), jnp.int32) for _ in range(3)],
            [pltpu.VMEM((CH, D), jnp.float32) for _ in range(2)],
            pltpu.VMEM((CT,), jnp.int32),
            pltpu.VMEM((CT,), jnp.int32),
            pltpu.VMEM((CT, D), jnp.float32),
            [pltpu.SemaphoreType.DMA for _ in range(2)],
            [pltpu.SemaphoreType.DMA for _ in range(2)],
            [pltpu.SemaphoreType.DMA for _ in range(3)],
            [pltpu.SemaphoreType.DMA for _ in range(3)],
            pltpu.SemaphoreType.DMA,
        ],
    )(qx, src, dst, zer)


_BLK = 2048
_NBLK = NP // _BLK


def _qparams(mn, mx):
    scale = jnp.maximum((mx - mn) / (_QMAX - _QMIN), 1e-8)
    zp = _QMIN - jnp.round(mn / scale)
    return scale, zp


def _fq(v, scale, zp):
    q = jnp.clip(jnp.round(v / scale) + zp, _QMIN, _QMAX)
    return (q - zp) * scale


def _finish_body(a0_ref, a1_ref, o_ref, smn, smx):
    p = pl.program_id(0)
    i = pl.program_id(1)
    t = a0_ref[...] + a1_ref[...]

    @pl.when((p == 0) & (i == 0))
    def _():
        smn[0] = jnp.min(t)
        smx[0] = jnp.max(t)

    @pl.when((p == 0) & (i > 0))
    def _():
        smn[0] = jnp.minimum(smn[0], jnp.min(t))
        smx[0] = jnp.maximum(smx[0], jnp.max(t))

    @pl.when(p == 1)
    def _():
        mn_s = smn[0]
        mx_s = smx[0]
        mn2 = jnp.minimum(mn_s, 0.0)
        mx2 = jnp.maximum(mx_s, 0.0)
        sc2, zp2 = _qparams(mn2, mx2)
        dq2 = _fq(t, sc2, zp2)
        mn3 = jnp.minimum(_fq(mn_s, sc2, zp2), 0.0)
        mx3 = jnp.maximum(_fq(mx_s, sc2, zp2), 0.0)
        sc3, zp3 = _qparams(mn3, mx3)
        o_ref[...] = _fq(dq2, sc3, zp3)


def _finish(agg2):
    return pl.pallas_call(
        _finish_body,
        grid=(2, _NBLK),
        in_specs=[
            pl.BlockSpec((_BLK, D), lambda p, i: (i, 0)),
            pl.BlockSpec((_BLK, D), lambda p, i: (i + _NBLK, 0)),
        ],
        out_specs=pl.BlockSpec((_BLK, D), lambda p, i: (i, 0)),
        out_shape=jax.ShapeDtypeStruct((NP, D), jnp.float32),
        scratch_shapes=[
            pltpu.SMEM((1,), jnp.float32),
            pltpu.SMEM((1,), jnp.float32),
        ],
    )(agg2, agg2)


def kernel(x, edge_index):
    src = edge_index[0].astype(jnp.int32)
    dst = edge_index[1].astype(jnp.int32)
    x = x.astype(jnp.float32)

    rmin, rmax = _rowminmax(x)
    omin, omax = _msg_minmax(rmin.reshape(N), rmax.reshape(N), src)
    qx = _quant_x(x, omin, omax)
    zer = jnp.zeros((RT, D), jnp.float32)
    agg2 = _scatter_agg(qx, src, dst, zer)
    return _finish(agg2)[:N]


# speedup vs baseline: 11.0072x; 1.1322x over previous
import functools

import jax
import jax.numpy as jnp
from jax import lax
from jax.experimental import pallas as pl
from jax.experimental.pallas import tpu as pltpu
from jax.experimental.pallas import tpu_sc as plsc

N = 10000
E = 320000
D = 128
NC = 2
NS = 16
NW = NC * NS
CH = 128
NFULL = 78
EW = NFULL * CH
ETAIL = E - NW * EW
NP = 10240
RT = NP // NS

_QMIN, _QMAX = -128.0, 127.0


def _mesh():
    return plsc.VectorSubcoreMesh(
        core_axis_name="c", subcore_axis_name="s", num_cores=NC,
        num_subcores=NS)


NR = 80


def _rowminmax_body(x_ref, mn_ref, mx_ref):
    pad = jnp.zeros((NR * 128 - N,), jnp.float32)
    mn = jnp.min(x_ref[...], axis=1)
    mx = jnp.max(x_ref[...], axis=1)
    mn_ref[...] = jnp.concatenate([mn, pad]).reshape(NR, 128)
    mx_ref[...] = jnp.concatenate([mx, pad]).reshape(NR, 128)


def _rowminmax(x):
    return pl.pallas_call(
        _rowminmax_body,
        out_shape=(
            jax.ShapeDtypeStruct((NR, 128), jnp.float32),
            jax.ShapeDtypeStruct((NR, 128), jnp.float32),
        ),
    )(x)


def _msg_minmax_body(rmin_hbm, rmax_hbm, ei_hbm, omin_hbm, omax_hbm,
                     rmin_v, rmax_v, idx_v, idt_v, tmn_v, tmx_v):
    c = lax.axis_index("c")
    s = lax.axis_index("s")
    wid = s * NC + c
    pltpu.sync_copy(rmin_hbm, rmin_v)
    pltpu.sync_copy(rmax_hbm, rmax_v)
    pltpu.sync_copy(ei_hbm.at[0, pl.ds(wid * EW, EW)], idx_v)
    pltpu.sync_copy(
        ei_hbm.at[0, pl.ds(NW * EW + (wid % 4) * CH, CH)], idt_v)

    def red(idx, carry):
        amn, amx = carry
        vr = lax.shift_right_logical(idx, 7)
        vc = lax.bitwise_and(idx, 127)
        vmn = plsc.load_gather(rmin_v, [vr, vc])
        vmx = plsc.load_gather(rmax_v, [vr, vc])
        return jnp.minimum(amn, vmn), jnp.maximum(amx, vmx)

    def body(i, carry):
        return red(idx_v[pl.ds(i * 16, 16)], carry)

    def bodyt(i, carry):
        return red(idt_v[pl.ds(i * 16, 16)], carry)

    init = (jnp.full((16,), jnp.inf, jnp.float32),
            jnp.full((16,), -jnp.inf, jnp.float32))
    carry = lax.fori_loop(0, EW // 16, body, init)
    amn, amx = lax.fori_loop(0, CH // 16, bodyt, carry)
    tmn_v[...] = amn
    tmx_v[...] = amx
    pltpu.sync_copy(tmn_v, omin_hbm.at[wid])
    pltpu.sync_copy(tmx_v, omax_hbm.at[wid])


def _msg_minmax(rmin, rmax, ei):
    return pl.kernel(
        _msg_minmax_body,
        out_type=(
            jax.ShapeDtypeStruct((NW, 16), jnp.float32),
            jax.ShapeDtypeStruct((NW, 16), jnp.float32),
        ),
        mesh=_mesh(),
        compiler_params=pltpu.CompilerParams(needs_layout_passes=False),
        scratch_types=[
            pltpu.VMEM((NR, 128), jnp.float32),
            pltpu.VMEM((NR, 128), jnp.float32),
            pltpu.VMEM((EW,), jnp.int32),
            pltpu.VMEM((CH,), jnp.int32),
            pltpu.VMEM((16,), jnp.float32),
            pltpu.VMEM((16,), jnp.float32),
        ],
    )(rmin, rmax, ei)


def _quant_x_body(x_ref, omin_ref, omax_ref, qx_ref):
    mn = jnp.minimum(jnp.min(omin_ref[...]), 0.0)
    mx = jnp.maximum(jnp.max(omax_ref[...]), 0.0)
    scale = jnp.maximum((mx - mn) / (_QMAX - _QMIN), 1e-8)
    zp = _QMIN - jnp.round(mn / scale)
    q = jnp.clip(jnp.round(x_ref[...] / scale) + zp, _QMIN, _QMAX)
    qx_ref[...] = (q - zp) * scale


def _quant_x(x, omin, omax):
    return pl.pallas_call(
        _quant_x_body,
        out_shape=jax.ShapeDtypeStruct((N, D), jnp.float32),
    )(x, omin, omax)


def _scatter_body(qx_hbm, ei_hbm, zer_hbm, out0_hbm, out1_hbm,
                  acc, srcb, dstb, rows, sg, ss, si, di, st):
    c = lax.axis_index("c")
    s = lax.axis_index("s")
    wid = s * NC + c
    ebase = wid * EW

    def start_idx(k, j):
        off = ebase + k * CH
        pltpu.async_copy(ei_hbm.at[0, pl.ds(off, CH)], srcb[j], si[j])
        pltpu.async_copy(ei_hbm.at[1, pl.ds(off, CH)], dstb[j], di[j])

    def wait_idx(j):
        pltpu.make_async_copy(ei_hbm.at[0, pl.ds(0, CH)], srcb[j], si[j]).wait()
        pltpu.make_async_copy(ei_hbm.at[0, pl.ds(0, CH)], dstb[j], di[j]).wait()

    def gather(b, j):
        pltpu.async_copy(qx_hbm.at[srcb[j]], rows[b], sg[b]).wait()

    def start_scatter(b, j):
        pltpu.async_copy(rows[b], acc.at[dstb[j]], ss[b], add=True)

    def wait_scatter(b):
        pltpu.make_async_copy(qx_hbm.at[pl.ds(0, CH)], rows[b], ss[b]).wait()

    def step(k, kk, prefetch=True):
        b, j = kk % 2, kk % 3
        wait_scatter(b)
        if prefetch:
            start_idx(k + 1, (j + 1) % 3)
        wait_idx(j)
        gather(b, j)
        start_scatter(b, j)

    pltpu.async_copy(zer_hbm, acc.at[pl.ds(s * RT, RT)], st)
    start_idx(0, 0)
    start_idx(1, 1)
    wait_idx(0)
    gather(0, 0)
    start_idx(2, 2)
    pltpu.make_async_copy(zer_hbm, acc.at[pl.ds(s * RT, RT)], st).wait()
    plsc.subcore_barrier()
    start_scatter(0, 0)
    wait_idx(1)
    gather(1, 1)
    start_scatter(1, 1)

    def body(g, carry):
        k0 = 2 + g * 6
        for i in range(6):
            step(k0 + i, 2 + i)
        return carry

    lax.fori_loop(0, 12, body, 0)

    for k in (74, 75, 76, 77):
        step(k, k, prefetch=(k + 1 < NFULL))
    wait_scatter(NFULL % 2)
    wait_scatter((NFULL + 1) % 2)

    @pl.when(wid < 4)
    def _():
        toff = NW * EW + wid * CH
        pltpu.sync_copy(ei_hbm.at[0, pl.ds(toff, CH)], srcb[0])
        pltpu.sync_copy(ei_hbm.at[1, pl.ds(toff, CH)], dstb[0])
        pltpu.async_copy(qx_hbm.at[srcb[0]], rows[0], st).wait()
        pltpu.sync_copy(rows[0], acc.at[dstb[0]], add=True)

    plsc.subcore_barrier()

    @pl.when(c == 0)
    def _():
        pltpu.sync_copy(acc.at[pl.ds(s * RT, RT)],
                        out0_hbm.at[pl.ds(s * RT, RT)])

    @pl.when(c == 1)
    def _():
        pltpu.sync_copy(acc.at[pl.ds(s * RT, RT)],
                        out1_hbm.at[pl.ds(s * RT, RT)])


def _scatter_agg(qx, ei, zer):
    return pl.kernel(
        _scatter_body,
        out_type=(
            jax.ShapeDtypeStruct((NP, D), jnp.float32),
            jax.ShapeDtypeStruct((NP, D), jnp.float32),
        ),
        mesh=_mesh(),
        scratch_types=[
            pltpu.VMEM_SHARED((NP, D), jnp.float32),
            [pltpu.VMEM((CH,), jnp.int32) for _ in range(3)],
            [pltpu.VMEM((CH,), jnp.int32) for _ in range(3)],
            [pltpu.VMEM((CH, D), jnp.float32) for _ in range(2)],
            [pltpu.SemaphoreType.DMA for _ in range(2)],
            [pltpu.SemaphoreType.DMA for _ in range(2)],
            [pltpu.SemaphoreType.DMA for _ in range(3)],
            [pltpu.SemaphoreType.DMA for _ in range(3)],
            pltpu.SemaphoreType.DMA,
        ],
    )(qx, ei, zer)


_BLK = 2000
_NBLK = N // _BLK


def _qparams(mn, mx):
    scale = jnp.maximum((mx - mn) / (_QMAX - _QMIN), 1e-8)
    zp = _QMIN - jnp.round(mn / scale)
    return scale, zp


def _fq(v, scale, zp):
    q = jnp.clip(jnp.round(v / scale) + zp, _QMIN, _QMAX)
    return (q - zp) * scale


def _finish_body(a0_ref, a1_ref, o_ref, smn, smx):
    p = pl.program_id(0)
    i = pl.program_id(1)
    t = a0_ref[...] + a1_ref[...]

    @pl.when((p == 0) & (i == 0))
    def _():
        smn[0] = jnp.min(t)
        smx[0] = jnp.max(t)

    @pl.when((p == 0) & (i > 0))
    def _():
        smn[0] = jnp.minimum(smn[0], jnp.min(t))
        smx[0] = jnp.maximum(smx[0], jnp.max(t))

    @pl.when(p == 1)
    def _():
        mn_s = smn[0]
        mx_s = smx[0]
        mn2 = jnp.minimum(mn_s, 0.0)
        mx2 = jnp.maximum(mx_s, 0.0)
        sc2, zp2 = _qparams(mn2, mx2)
        dq2 = _fq(t, sc2, zp2)
        mn3 = jnp.minimum(_fq(mn_s, sc2, zp2), 0.0)
        mx3 = jnp.maximum(_fq(mx_s, sc2, zp2), 0.0)
        sc3, zp3 = _qparams(mn3, mx3)
        o_ref[...] = _fq(dq2, sc3, zp3)


def _finish(a0, a1):
    return pl.pallas_call(
        _finish_body,
        grid=(2, _NBLK),
        in_specs=[
            pl.BlockSpec((_BLK, D), lambda p, i: (i, 0)),
            pl.BlockSpec((_BLK, D), lambda p, i: (i, 0)),
        ],
        out_specs=pl.BlockSpec((_BLK, D), lambda p, i: (i, 0)),
        out_shape=jax.ShapeDtypeStruct((N, D), jnp.float32),
        scratch_shapes=[
            pltpu.SMEM((1,), jnp.float32),
            pltpu.SMEM((1,), jnp.float32),
        ],
    )(a0, a1)


def kernel(x, edge_index):
    ei = edge_index.astype(jnp.int32)
    x = x.astype(jnp.float32)

    rmin, rmax = _rowminmax(x)
    omin, omax = _msg_minmax(rmin, rmax, ei)
    qx = _quant_x(x, omin, omax)
    zer = jnp.zeros((RT, D), jnp.float32)
    a0, a1 = _scatter_agg(qx, ei, zer)
    return _finish(a0, a1)


# speedup vs baseline: 12.5163x; 1.1371x over previous
import functools

import jax
import jax.numpy as jnp
from jax import lax
from jax.experimental import pallas as pl
from jax.experimental.pallas import tpu as pltpu
from jax.experimental.pallas import tpu_sc as plsc

N = 10000
E = 320000
D = 128
NC = 2
NS = 16
NW = NC * NS
CH = 128
NFULL = 78
EW = NFULL * CH
ETAIL = E - NW * EW
NP = 10240
RT = NP // NS

_QMIN, _QMAX = -128.0, 127.0


def _mesh():
    return plsc.VectorSubcoreMesh(
        core_axis_name="c", subcore_axis_name="s", num_cores=NC,
        num_subcores=NS)


NR = 80


def _rowminmax_body(x_ref, mn_ref, mx_ref):
    pad = jnp.zeros((NR * 128 - N,), jnp.float32)
    mn = jnp.min(x_ref[...], axis=1)
    mx = jnp.max(x_ref[...], axis=1)
    mn_ref[...] = jnp.concatenate([mn, pad]).reshape(NR, 128)
    mx_ref[...] = jnp.concatenate([mx, pad]).reshape(NR, 128)


def _rowminmax(x):
    return pl.pallas_call(
        _rowminmax_body,
        out_shape=(
            jax.ShapeDtypeStruct((NR, 128), jnp.float32),
            jax.ShapeDtypeStruct((NR, 128), jnp.float32),
        ),
    )(x)


def _msg_minmax_body(rmin_hbm, rmax_hbm, ei_hbm, omin_hbm, omax_hbm,
                     rmin_v, rmax_v, idx_v, idt_v, tmn_v, tmx_v):
    c = lax.axis_index("c")
    s = lax.axis_index("s")
    wid = s * NC + c
    pltpu.sync_copy(rmin_hbm, rmin_v)
    pltpu.sync_copy(rmax_hbm, rmax_v)
    pltpu.sync_copy(ei_hbm.at[0, pl.ds(wid * EW, EW)], idx_v)
    pltpu.sync_copy(
        ei_hbm.at[0, pl.ds(NW * EW + (wid % 4) * CH, CH)], idt_v)

    def red(idx, carry):
        amn, amx = carry
        vr = lax.shift_right_logical(idx, 7)
        vc = lax.bitwise_and(idx, 127)
        vmn = plsc.load_gather(rmin_v, [vr, vc])
        vmx = plsc.load_gather(rmax_v, [vr, vc])
        return jnp.minimum(amn, vmn), jnp.maximum(amx, vmx)

    def body(i, carry):
        return red(idx_v[pl.ds(i * 16, 16)], carry)

    def bodyt(i, carry):
        return red(idt_v[pl.ds(i * 16, 16)], carry)

    init = (jnp.full((16,), jnp.inf, jnp.float32),
            jnp.full((16,), -jnp.inf, jnp.float32))
    carry = lax.fori_loop(0, EW // 16, body, init)
    amn, amx = lax.fori_loop(0, CH // 16, bodyt, carry)
    tmn_v[...] = amn
    tmx_v[...] = amx
    pltpu.sync_copy(tmn_v, omin_hbm.at[wid])
    pltpu.sync_copy(tmx_v, omax_hbm.at[wid])


def _msg_minmax(rmin, rmax, ei):
    return pl.kernel(
        _msg_minmax_body,
        out_type=(
            jax.ShapeDtypeStruct((NW, 16), jnp.float32),
            jax.ShapeDtypeStruct((NW, 16), jnp.float32),
        ),
        mesh=_mesh(),
        compiler_params=pltpu.CompilerParams(needs_layout_passes=False),
        scratch_types=[
            pltpu.VMEM((NR, 128), jnp.float32),
            pltpu.VMEM((NR, 128), jnp.float32),
            pltpu.VMEM((EW,), jnp.int32),
            pltpu.VMEM((CH,), jnp.int32),
            pltpu.VMEM((16,), jnp.float32),
            pltpu.VMEM((16,), jnp.float32),
        ],
    )(rmin, rmax, ei)


def _quant_x_body(x_ref, omin_ref, omax_ref, qx_ref):
    mn = jnp.minimum(jnp.min(omin_ref[...]), 0.0)
    mx = jnp.maximum(jnp.max(omax_ref[...]), 0.0)
    scale = jnp.maximum((mx - mn) / (_QMAX - _QMIN), 1e-8)
    zp = _QMIN - jnp.round(mn / scale)
    q = jnp.clip(jnp.round(x_ref[...] / scale) + zp, _QMIN, _QMAX)
    qx_ref[...] = (q - zp) * scale


def _quant_x(x, omin, omax):
    return pl.pallas_call(
        _quant_x_body,
        out_shape=jax.ShapeDtypeStruct((N, D), jnp.float32),
    )(x, omin, omax)


def _scatter_body(qx_hbm, ei_hbm, zer_hbm, out0_hbm, out1_hbm,
                  acc, srcb, dstb, rows, sg, ss, si, di, st):
    c = lax.axis_index("c")
    s = lax.axis_index("s")
    wid = s * NC + c
    ebase = wid * EW

    def start_idx(k, j):
        off = ebase + k * CH
        pltpu.async_copy(ei_hbm.at[0, pl.ds(off, CH)], srcb[j], si[j])
        pltpu.async_copy(ei_hbm.at[1, pl.ds(off, CH)], dstb[j], di[j])

    def wait_idx(j):
        pltpu.make_async_copy(ei_hbm.at[0, pl.ds(0, CH)], srcb[j], si[j]).wait()
        pltpu.make_async_copy(ei_hbm.at[0, pl.ds(0, CH)], dstb[j], di[j]).wait()

    def start_gather(b, j):
        pltpu.async_copy(qx_hbm.at[srcb[j]], rows[b], sg[b])

    def wait_gather(b):
        pltpu.make_async_copy(qx_hbm.at[pl.ds(0, CH)], rows[b], sg[b]).wait()

    def start_scatter(b, j):
        pltpu.async_copy(rows[b], acc.at[dstb[j]], ss[b], add=True)

    def wait_scatter(b):
        pltpu.make_async_copy(qx_hbm.at[pl.ds(0, CH)], rows[b], ss[b]).wait()

    def step(k, kk, has_idx=True, has_gather=True):
        b, j = kk % 2, kk % 3
        wait_scatter(1 - b)
        if has_idx:
            start_idx(k + 2, (kk + 2) % 3)
        if has_gather:
            wait_idx((j + 1) % 3)
            start_gather(1 - b, (j + 1) % 3)
        wait_gather(b)
        start_scatter(b, j)

    pltpu.async_copy(zer_hbm, acc.at[pl.ds(s * RT, RT)], st)
    start_idx(0, 0)
    start_idx(1, 1)
    wait_idx(0)
    start_gather(0, 0)
    pltpu.make_async_copy(zer_hbm, acc.at[pl.ds(s * RT, RT)], st).wait()
    plsc.subcore_barrier()
    start_idx(2, 2)
    wait_idx(1)
    start_gather(1, 1)
    wait_gather(0)
    start_scatter(0, 0)

    def body(g, carry):
        k0 = 1 + g * 6
        for i in range(6):
            step(k0 + i, 1 + i)
        return carry

    lax.fori_loop(0, 12, body, 0)

    for k in (73, 74, 75, 76, 77):
        step(k, k, has_idx=(k + 2 < NFULL), has_gather=(k + 1 < NFULL))
    wait_scatter((NFULL - 1) % 2)

    @pl.when(wid < 4)
    def _():
        toff = NW * EW + wid * CH
        pltpu.sync_copy(ei_hbm.at[0, pl.ds(toff, CH)], srcb[0])
        pltpu.sync_copy(ei_hbm.at[1, pl.ds(toff, CH)], dstb[0])
        pltpu.async_copy(qx_hbm.at[srcb[0]], rows[0], st).wait()
        pltpu.sync_copy(rows[0], acc.at[dstb[0]], add=True)

    plsc.subcore_barrier()

    @pl.when(c == 0)
    def _():
        pltpu.sync_copy(acc.at[pl.ds(s * RT, RT)],
                        out0_hbm.at[pl.ds(s * RT, RT)])

    @pl.when(c == 1)
    def _():
        pltpu.sync_copy(acc.at[pl.ds(s * RT, RT)],
                        out1_hbm.at[pl.ds(s * RT, RT)])


def _scatter_agg(qx, ei, zer):
    return pl.kernel(
        _scatter_body,
        out_type=(
            jax.ShapeDtypeStruct((NP, D), jnp.float32),
            jax.ShapeDtypeStruct((NP, D), jnp.float32),
        ),
        mesh=_mesh(),
        scratch_types=[
            pltpu.VMEM_SHARED((NP, D), jnp.float32),
            [pltpu.VMEM((CH,), jnp.int32) for _ in range(3)],
            [pltpu.VMEM((CH,), jnp.int32) for _ in range(3)],
            [pltpu.VMEM((CH, D), jnp.float32) for _ in range(2)],
            [pltpu.SemaphoreType.DMA for _ in range(2)],
            [pltpu.SemaphoreType.DMA for _ in range(2)],
            [pltpu.SemaphoreType.DMA for _ in range(3)],
            [pltpu.SemaphoreType.DMA for _ in range(3)],
            pltpu.SemaphoreType.DMA,
        ],
    )(qx, ei, zer)


_BLK = 2000
_NBLK = N // _BLK


def _qparams(mn, mx):
    scale = jnp.maximum((mx - mn) / (_QMAX - _QMIN), 1e-8)
    zp = _QMIN - jnp.round(mn / scale)
    return scale, zp


def _fq(v, scale, zp):
    q = jnp.clip(jnp.round(v / scale) + zp, _QMIN, _QMAX)
    return (q - zp) * scale


def _finish_body(a0_ref, a1_ref, o_ref, smn, smx):
    p = pl.program_id(0)
    i = pl.program_id(1)
    t = a0_ref[...] + a1_ref[...]

    @pl.when((p == 0) & (i == 0))
    def _():
        smn[0] = jnp.min(t)
        smx[0] = jnp.max(t)

    @pl.when((p == 0) & (i > 0))
    def _():
        smn[0] = jnp.minimum(smn[0], jnp.min(t))
        smx[0] = jnp.maximum(smx[0], jnp.max(t))

    @pl.when(p == 1)
    def _():
        mn_s = smn[0]
        mx_s = smx[0]
        mn2 = jnp.minimum(mn_s, 0.0)
        mx2 = jnp.maximum(mx_s, 0.0)
        sc2, zp2 = _qparams(mn2, mx2)
        dq2 = _fq(t, sc2, zp2)
        mn3 = jnp.minimum(_fq(mn_s, sc2, zp2), 0.0)
        mx3 = jnp.maximum(_fq(mx_s, sc2, zp2), 0.0)
        sc3, zp3 = _qparams(mn3, mx3)
        o_ref[...] = _fq(dq2, sc3, zp3)


def _finish(a0, a1):
    return pl.pallas_call(
        _finish_body,
        grid=(2, _NBLK),
        in_specs=[
            pl.BlockSpec((_BLK, D), lambda p, i: (i, 0)),
            pl.BlockSpec((_BLK, D), lambda p, i: (i, 0)),
        ],
        out_specs=pl.BlockSpec((_BLK, D), lambda p, i: (i, 0)),
        out_shape=jax.ShapeDtypeStruct((N, D), jnp.float32),
        scratch_shapes=[
            pltpu.SMEM((1,), jnp.float32),
            pltpu.SMEM((1,), jnp.float32),
        ],
    )(a0, a1)


def kernel(x, edge_index):
    ei = edge_index.astype(jnp.int32)
    x = x.astype(jnp.float32)

    rmin, rmax = _rowminmax(x)
    omin, omax = _msg_minmax(rmin, rmax, ei)
    qx = _quant_x(x, omin, omax)
    zer = jnp.zeros((RT, D), jnp.float32)
    a0, a1 = _scatter_agg(qx, ei, zer)
    return _finish(a0, a1)


# speedup vs baseline: 12.5548x; 1.0031x over previous
import functools

import jax
import jax.numpy as jnp
from jax import lax
from jax.experimental import pallas as pl
from jax.experimental.pallas import tpu as pltpu
from jax.experimental.pallas import tpu_sc as plsc

N = 10000
E = 320000
D = 128
NC = 2
NS = 16
NW = NC * NS
CH = 128
NFULL = 78
EW = NFULL * CH
ETAIL = E - NW * EW
NP = 10240
RT = NP // NS

_QMIN, _QMAX = -128.0, 127.0


def _mesh():
    return plsc.VectorSubcoreMesh(
        core_axis_name="c", subcore_axis_name="s", num_cores=NC,
        num_subcores=NS)


NR = 80


def _rowminmax_body(x_ref, mn_ref, mx_ref):
    pad = jnp.zeros((NR * 128 - N,), jnp.float32)
    mn = jnp.min(x_ref[...], axis=1)
    mx = jnp.max(x_ref[...], axis=1)
    mn_ref[...] = jnp.concatenate([mn, pad]).reshape(NR, 128)
    mx_ref[...] = jnp.concatenate([mx, pad]).reshape(NR, 128)


def _rowminmax(x):
    return pl.pallas_call(
        _rowminmax_body,
        out_shape=(
            jax.ShapeDtypeStruct((NR, 128), jnp.float32),
            jax.ShapeDtypeStruct((NR, 128), jnp.float32),
        ),
    )(x)


def _msg_minmax_body(rmin_hbm, rmax_hbm, ei_hbm, omin_hbm, omax_hbm,
                     rmin_v, rmax_v, idx_v, idt_v, tmn_v, tmx_v):
    c = lax.axis_index("c")
    s = lax.axis_index("s")
    wid = s * NC + c
    pltpu.sync_copy(rmin_hbm, rmin_v)
    pltpu.sync_copy(rmax_hbm, rmax_v)
    pltpu.sync_copy(ei_hbm.at[0, pl.ds(wid * EW, EW)], idx_v)
    pltpu.sync_copy(
        ei_hbm.at[0, pl.ds(NW * EW + (wid % 4) * CH, CH)], idt_v)

    def red(idx, carry):
        amn, amx = carry
        vr = lax.shift_right_logical(idx, 7)
        vc = lax.bitwise_and(idx, 127)
        vmn = plsc.load_gather(rmin_v, [vr, vc])
        vmx = plsc.load_gather(rmax_v, [vr, vc])
        return jnp.minimum(amn, vmn), jnp.maximum(amx, vmx)

    def body(i, carry):
        return red(idx_v[pl.ds(i * 16, 16)], carry)

    def bodyt(i, carry):
        return red(idt_v[pl.ds(i * 16, 16)], carry)

    init = (jnp.full((16,), jnp.inf, jnp.float32),
            jnp.full((16,), -jnp.inf, jnp.float32))
    carry = lax.fori_loop(0, EW // 16, body, init)
    amn, amx = lax.fori_loop(0, CH // 16, bodyt, carry)
    tmn_v[...] = amn
    tmx_v[...] = amx
    pltpu.sync_copy(tmn_v, omin_hbm.at[wid])
    pltpu.sync_copy(tmx_v, omax_hbm.at[wid])


def _msg_minmax(rmin, rmax, ei):
    return pl.kernel(
        _msg_minmax_body,
        out_type=(
            jax.ShapeDtypeStruct((NW, 16), jnp.float32),
            jax.ShapeDtypeStruct((NW, 16), jnp.float32),
        ),
        mesh=_mesh(),
        compiler_params=pltpu.CompilerParams(needs_layout_passes=False),
        scratch_types=[
            pltpu.VMEM((NR, 128), jnp.float32),
            pltpu.VMEM((NR, 128), jnp.float32),
            pltpu.VMEM((EW,), jnp.int32),
            pltpu.VMEM((CH,), jnp.int32),
            pltpu.VMEM((16,), jnp.float32),
            pltpu.VMEM((16,), jnp.float32),
        ],
    )(rmin, rmax, ei)


def _quant_x_body(x_ref, omin_ref, omax_ref, qx_ref):
    mn = jnp.minimum(jnp.min(omin_ref[...]), 0.0)
    mx = jnp.maximum(jnp.max(omax_ref[...]), 0.0)
    scale = jnp.maximum((mx - mn) / (_QMAX - _QMIN), 1e-8)
    zp = _QMIN - jnp.round(mn / scale)
    q = jnp.clip(jnp.round(x_ref[...] / scale) + zp, _QMIN, _QMAX)
    qx_ref[...] = (q - zp) * scale


def _quant_x(x, omin, omax):
    return pl.pallas_call(
        _quant_x_body,
        out_shape=jax.ShapeDtypeStruct((N, D), jnp.float32),
    )(x, omin, omax)


def _scatter_body(qx_hbm, ei_hbm, zer_hbm, out0_hbm, out1_hbm,
                  acc, srcb, dstb, rows, sg, ss, si, di, st):
    c = lax.axis_index("c")
    s = lax.axis_index("s")
    wid = s * NC + c
    ebase = wid * EW

    def start_idx(k, j):
        off = ebase + k * CH
        pltpu.async_copy(ei_hbm.at[0, pl.ds(off, CH)], srcb[j], si[j])
        pltpu.async_copy(ei_hbm.at[1, pl.ds(off, CH)], dstb[j], di[j])

    def wait_idx(j):
        pltpu.make_async_copy(ei_hbm.at[0, pl.ds(0, CH)], srcb[j], si[j]).wait()
        pltpu.make_async_copy(ei_hbm.at[0, pl.ds(0, CH)], dstb[j], di[j]).wait()

    H = CH // 2

    def start_gather(b, j):
        pltpu.async_copy(qx_hbm.at[srcb[j].at[pl.ds(0, H)]],
                         rows[b].at[pl.ds(0, H)], sg[b])
        pltpu.async_copy(qx_hbm.at[srcb[j].at[pl.ds(H, H)]],
                         rows[b].at[pl.ds(H, H)], sg[b])

    def wait_gather(b):
        pltpu.make_async_copy(qx_hbm.at[pl.ds(0, CH)], rows[b], sg[b]).wait()

    def start_scatter(b, j):
        pltpu.async_copy(rows[b], acc.at[dstb[j]], ss[b], add=True)

    def wait_scatter(b):
        pltpu.make_async_copy(qx_hbm.at[pl.ds(0, CH)], rows[b], ss[b]).wait()

    def step(k, kk, has_idx=True, has_gather=True):
        b, j = kk % 2, kk % 3
        wait_scatter(1 - b)
        if has_idx:
            start_idx(k + 2, (kk + 2) % 3)
        if has_gather:
            wait_idx((j + 1) % 3)
            start_gather(1 - b, (j + 1) % 3)
        wait_gather(b)
        start_scatter(b, j)

    pltpu.async_copy(zer_hbm, acc.at[pl.ds(s * RT, RT)], st)
    start_idx(0, 0)
    start_idx(1, 1)
    wait_idx(0)
    start_gather(0, 0)
    pltpu.make_async_copy(zer_hbm, acc.at[pl.ds(s * RT, RT)], st).wait()
    plsc.subcore_barrier()
    start_idx(2, 2)
    wait_idx(1)
    start_gather(1, 1)
    wait_gather(0)
    start_scatter(0, 0)

    def body(g, carry):
        k0 = 1 + g * 6
        for i in range(6):
            step(k0 + i, 1 + i)
        return carry

    lax.fori_loop(0, 12, body, 0)

    for k in (73, 74, 75, 76, 77):
        step(k, k, has_idx=(k + 2 < NFULL), has_gather=(k + 1 < NFULL))
    wait_scatter((NFULL - 1) % 2)

    @pl.when(wid < 4)
    def _():
        toff = NW * EW + wid * CH
        pltpu.sync_copy(ei_hbm.at[0, pl.ds(toff, CH)], srcb[0])
        pltpu.sync_copy(ei_hbm.at[1, pl.ds(toff, CH)], dstb[0])
        pltpu.async_copy(qx_hbm.at[srcb[0]], rows[0], st).wait()
        pltpu.sync_copy(rows[0], acc.at[dstb[0]], add=True)

    plsc.subcore_barrier()

    @pl.when(c == 0)
    def _():
        pltpu.sync_copy(acc.at[pl.ds(s * RT, RT)],
                        out0_hbm.at[pl.ds(s * RT, RT)])

    @pl.when(c == 1)
    def _():
        pltpu.sync_copy(acc.at[pl.ds(s * RT, RT)],
                        out1_hbm.at[pl.ds(s * RT, RT)])


def _scatter_agg(qx, ei, zer):
    return pl.kernel(
        _scatter_body,
        out_type=(
            jax.ShapeDtypeStruct((NP, D), jnp.float32),
            jax.ShapeDtypeStruct((NP, D), jnp.float32),
        ),
        mesh=_mesh(),
        scratch_types=[
            pltpu.VMEM_SHARED((NP, D), jnp.float32),
            [pltpu.VMEM((CH,), jnp.int32) for _ in range(3)],
            [pltpu.VMEM((CH,), jnp.int32) for _ in range(3)],
            [pltpu.VMEM((CH, D), jnp.float32) for _ in range(2)],
            [pltpu.SemaphoreType.DMA for _ in range(2)],
            [pltpu.SemaphoreType.DMA for _ in range(2)],
            [pltpu.SemaphoreType.DMA for _ in range(3)],
            [pltpu.SemaphoreType.DMA for _ in range(3)],
            pltpu.SemaphoreType.DMA,
        ],
    )(qx, ei, zer)


_BLK = 2000
_NBLK = N // _BLK


def _qparams(mn, mx):
    scale = jnp.maximum((mx - mn) / (_QMAX - _QMIN), 1e-8)
    zp = _QMIN - jnp.round(mn / scale)
    return scale, zp


def _fq(v, scale, zp):
    q = jnp.clip(jnp.round(v / scale) + zp, _QMIN, _QMAX)
    return (q - zp) * scale


def _finish_body(a0_ref, a1_ref, o_ref, smn, smx):
    p = pl.program_id(0)
    i = pl.program_id(1)
    t = a0_ref[...] + a1_ref[...]

    @pl.when((p == 0) & (i == 0))
    def _():
        smn[0] = jnp.min(t)
        smx[0] = jnp.max(t)

    @pl.when((p == 0) & (i > 0))
    def _():
        smn[0] = jnp.minimum(smn[0], jnp.min(t))
        smx[0] = jnp.maximum(smx[0], jnp.max(t))

    @pl.when(p == 1)
    def _():
        mn_s = smn[0]
        mx_s = smx[0]
        mn2 = jnp.minimum(mn_s, 0.0)
        mx2 = jnp.maximum(mx_s, 0.0)
        sc2, zp2 = _qparams(mn2, mx2)
        dq2 = _fq(t, sc2, zp2)
        mn3 = jnp.minimum(_fq(mn_s, sc2, zp2), 0.0)
        mx3 = jnp.maximum(_fq(mx_s, sc2, zp2), 0.0)
        sc3, zp3 = _qparams(mn3, mx3)
        o_ref[...] = _fq(dq2, sc3, zp3)


def _finish(a0, a1):
    return pl.pallas_call(
        _finish_body,
        grid=(2, _NBLK),
        in_specs=[
            pl.BlockSpec((_BLK, D), lambda p, i: (i, 0)),
            pl.BlockSpec((_BLK, D), lambda p, i: (i, 0)),
        ],
        out_specs=pl.BlockSpec((_BLK, D), lambda p, i: (i, 0)),
        out_shape=jax.ShapeDtypeStruct((N, D), jnp.float32),
        scratch_shapes=[
            pltpu.SMEM((1,), jnp.float32),
            pltpu.SMEM((1,), jnp.float32),
        ],
    )(a0, a1)


def kernel(x, edge_index):
    ei = edge_index.astype(jnp.int32)
    x = x.astype(jnp.float32)

    rmin, rmax = _rowminmax(x)
    omin, omax = _msg_minmax(rmin, rmax, ei)
    qx = _quant_x(x, omin, omax)
    zer = jnp.zeros((RT, D), jnp.float32)
    a0, a1 = _scatter_agg(qx, ei, zer)
    return _finish(a0, a1)


# speedup vs baseline: 13.1347x; 1.0462x over previous
import functools

import jax
import jax.numpy as jnp
from jax import lax
from jax.experimental import pallas as pl
from jax.experimental.pallas import tpu as pltpu
from jax.experimental.pallas import tpu_sc as plsc

N = 10000
E = 320000
D = 128
NC = 2
NS = 16
NW = NC * NS
CH = 128
NFULL = 78
EW = NFULL * CH
ETAIL = E - NW * EW
NP = 10112
RT = NP // NS

_QMIN, _QMAX = -128.0, 127.0


def _mesh():
    return plsc.VectorSubcoreMesh(
        core_axis_name="c", subcore_axis_name="s", num_cores=NC,
        num_subcores=NS)


NR = 80


def _rowminmax_body(x_ref, mn_ref, mx_ref):
    pad = jnp.zeros((NR * 128 - N,), jnp.float32)
    mn = jnp.min(x_ref[...], axis=1)
    mx = jnp.max(x_ref[...], axis=1)
    mn_ref[...] = jnp.concatenate([mn, pad]).reshape(NR, 128)
    mx_ref[...] = jnp.concatenate([mx, pad]).reshape(NR, 128)


def _rowminmax(x):
    return pl.pallas_call(
        _rowminmax_body,
        out_shape=(
            jax.ShapeDtypeStruct((NR, 128), jnp.float32),
            jax.ShapeDtypeStruct((NR, 128), jnp.float32),
        ),
    )(x)


def _msg_minmax_body(rmin_hbm, rmax_hbm, ei_hbm, omin_hbm, omax_hbm,
                     rmin_v, rmax_v, idx_v, idt_v, tmn_v, tmx_v):
    c = lax.axis_index("c")
    s = lax.axis_index("s")
    wid = s * NC + c
    pltpu.sync_copy(rmin_hbm, rmin_v)
    pltpu.sync_copy(rmax_hbm, rmax_v)
    pltpu.sync_copy(ei_hbm.at[0, pl.ds(wid * EW, EW)], idx_v)
    pltpu.sync_copy(
        ei_hbm.at[0, pl.ds(NW * EW + (wid % 4) * CH, CH)], idt_v)

    def red(idx, carry):
        amn, amx = carry
        vr = lax.shift_right_logical(idx, 7)
        vc = lax.bitwise_and(idx, 127)
        vmn = plsc.load_gather(rmin_v, [vr, vc])
        vmx = plsc.load_gather(rmax_v, [vr, vc])
        return jnp.minimum(amn, vmn), jnp.maximum(amx, vmx)

    def body(i, carry):
        return red(idx_v[pl.ds(i * 16, 16)], carry)

    def bodyt(i, carry):
        return red(idt_v[pl.ds(i * 16, 16)], carry)

    init = (jnp.full((16,), jnp.inf, jnp.float32),
            jnp.full((16,), -jnp.inf, jnp.float32))
    carry = lax.fori_loop(0, EW // 16, body, init)
    amn, amx = lax.fori_loop(0, CH // 16, bodyt, carry)
    tmn_v[...] = amn
    tmx_v[...] = amx
    pltpu.sync_copy(tmn_v, omin_hbm.at[wid])
    pltpu.sync_copy(tmx_v, omax_hbm.at[wid])


def _msg_minmax(rmin, rmax, ei):
    return pl.kernel(
        _msg_minmax_body,
        out_type=(
            jax.ShapeDtypeStruct((NW, 16), jnp.float32),
            jax.ShapeDtypeStruct((NW, 16), jnp.float32),
        ),
        mesh=_mesh(),
        compiler_params=pltpu.CompilerParams(needs_layout_passes=False),
        scratch_types=[
            pltpu.VMEM((NR, 128), jnp.float32),
            pltpu.VMEM((NR, 128), jnp.float32),
            pltpu.VMEM((EW,), jnp.int32),
            pltpu.VMEM((CH,), jnp.int32),
            pltpu.VMEM((16,), jnp.float32),
            pltpu.VMEM((16,), jnp.float32),
        ],
    )(rmin, rmax, ei)


def _quant_x_body(x_ref, omin_ref, omax_ref, qx_ref):
    mn = jnp.minimum(jnp.min(omin_ref[...]), 0.0)
    mx = jnp.maximum(jnp.max(omax_ref[...]), 0.0)
    scale = jnp.maximum((mx - mn) / (_QMAX - _QMIN), 1e-8)
    zp = _QMIN - jnp.round(mn / scale)
    q = jnp.clip(jnp.round(x_ref[...] / scale) + zp, _QMIN, _QMAX)
    qx_ref[...] = (q - zp) * scale


def _quant_x(x, omin, omax):
    return pl.pallas_call(
        _quant_x_body,
        out_shape=jax.ShapeDtypeStruct((N, D), jnp.float32),
    )(x, omin, omax)


def _scatter_body(qx_hbm, ei_hbm, zer_hbm, out0_hbm, out1_hbm,
                  acc, srcb, dstb, rows, sg, ss, si, di, st):
    c = lax.axis_index("c")
    s = lax.axis_index("s")
    wid = s * NC + c
    ebase = wid * EW

    def start_idx(k, kk2):
        off = ebase + k * CH
        pltpu.async_copy(ei_hbm.at[0, pl.ds(off, CH)], srcb[kk2 % 3],
                         si[kk2 % 3])
        pltpu.async_copy(ei_hbm.at[1, pl.ds(off, CH)], dstb[kk2 % 4],
                         di[kk2 % 4])

    def wait_idx(kk2):
        pltpu.make_async_copy(ei_hbm.at[0, pl.ds(0, CH)], srcb[kk2 % 3],
                              si[kk2 % 3]).wait()
        pltpu.make_async_copy(ei_hbm.at[0, pl.ds(0, CH)], dstb[kk2 % 4],
                              di[kk2 % 4]).wait()

    H = CH // 2

    def start_gather(b, js):
        pltpu.async_copy(qx_hbm.at[srcb[js].at[pl.ds(0, H)]],
                         rows[b].at[pl.ds(0, H)], sg[b])
        pltpu.async_copy(qx_hbm.at[srcb[js].at[pl.ds(H, H)]],
                         rows[b].at[pl.ds(H, H)], sg[b])

    def wait_gather(b):
        pltpu.make_async_copy(qx_hbm.at[pl.ds(0, CH)], rows[b], sg[b]).wait()

    def start_scatter(b, jd):
        pltpu.async_copy(rows[b], acc.at[dstb[jd]], ss[b], add=True)

    def wait_scatter(b):
        pltpu.make_async_copy(qx_hbm.at[pl.ds(0, CH)], rows[b], ss[b]).wait()

    def step(k, kk, has_wait=True, has_idx=True, has_gather=True):
        b = kk % 3
        if has_wait:
            wait_scatter((kk + 1) % 3)
        if has_idx:
            start_idx(k + 2, kk + 2)
        if has_gather:
            wait_idx(kk + 1)
            start_gather((kk + 1) % 3, (kk + 1) % 3)
        wait_gather(b)
        start_scatter(b, kk % 4)

    pltpu.async_copy(zer_hbm, acc.at[pl.ds(s * RT, RT)], st)
    start_idx(0, 0)
    start_idx(1, 1)
    wait_idx(0)
    start_gather(0, 0)
    pltpu.make_async_copy(zer_hbm, acc.at[pl.ds(s * RT, RT)], st).wait()
    plsc.subcore_barrier()
    step(0, 0, has_wait=False)
    step(1, 1, has_wait=False)

    def body(g, carry):
        k0 = 2 + g * 12
        for i in range(12):
            step(k0 + i, 2 + i)
        return carry

    lax.fori_loop(0, 6, body, 0)

    for k in (74, 75, 76, 77):
        step(k, k, has_idx=(k + 2 < NFULL), has_gather=(k + 1 < NFULL))
    wait_scatter((NFULL - 2) % 3)
    wait_scatter((NFULL - 1) % 3)

    @pl.when(wid < 4)
    def _():
        toff = NW * EW + wid * CH
        pltpu.sync_copy(ei_hbm.at[0, pl.ds(toff, CH)], srcb[0])
        pltpu.sync_copy(ei_hbm.at[1, pl.ds(toff, CH)], dstb[0])
        pltpu.async_copy(qx_hbm.at[srcb[0]], rows[0], st).wait()
        pltpu.sync_copy(rows[0], acc.at[dstb[0]], add=True)

    plsc.subcore_barrier()

    @pl.when(c == 0)
    def _():
        pltpu.sync_copy(acc.at[pl.ds(s * RT, RT)],
                        out0_hbm.at[pl.ds(s * RT, RT)])

    @pl.when(c == 1)
    def _():
        pltpu.sync_copy(acc.at[pl.ds(s * RT, RT)],
                        out1_hbm.at[pl.ds(s * RT, RT)])


def _scatter_agg(qx, ei, zer):
    return pl.kernel(
        _scatter_body,
        out_type=(
            jax.ShapeDtypeStruct((NP, D), jnp.float32),
            jax.ShapeDtypeStruct((NP, D), jnp.float32),
        ),
        mesh=_mesh(),
        scratch_types=[
            pltpu.VMEM_SHARED((NP, D), jnp.float32),
            [pltpu.VMEM((CH,), jnp.int32) for _ in range(3)],
            [pltpu.VMEM((CH,), jnp.int32) for _ in range(4)],
            [pltpu.VMEM((CH, D), jnp.float32) for _ in range(3)],
            [pltpu.SemaphoreType.DMA for _ in range(3)],
            [pltpu.SemaphoreType.DMA for _ in range(3)],
            [pltpu.SemaphoreType.DMA for _ in range(3)],
            [pltpu.SemaphoreType.DMA for _ in range(4)],
            pltpu.SemaphoreType.DMA,
        ],
    )(qx, ei, zer)


_BLK = 2000
_NBLK = N // _BLK


def _qparams(mn, mx):
    scale = jnp.maximum((mx - mn) / (_QMAX - _QMIN), 1e-8)
    zp = _QMIN - jnp.round(mn / scale)
    return scale, zp


def _fq(v, scale, zp):
    q = jnp.clip(jnp.round(v / scale) + zp, _QMIN, _QMAX)
    return (q - zp) * scale


def _finish_body(a0_ref, a1_ref, o_ref, smn, smx):
    p = pl.program_id(0)
    i = pl.program_id(1)
    t = a0_ref[...] + a1_ref[...]

    @pl.when((p == 0) & (i == 0))
    def _():
        smn[0] = jnp.min(t)
        smx[0] = jnp.max(t)

    @pl.when((p == 0) & (i > 0))
    def _():
        smn[0] = jnp.minimum(smn[0], jnp.min(t))
        smx[0] = jnp.maximum(smx[0], jnp.max(t))

    @pl.when(p == 1)
    def _():
        mn_s = smn[0]
        mx_s = smx[0]
        mn2 = jnp.minimum(mn_s, 0.0)
        mx2 = jnp.maximum(mx_s, 0.0)
        sc2, zp2 = _qparams(mn2, mx2)
        dq2 = _fq(t, sc2, zp2)
        mn3 = jnp.minimum(_fq(mn_s, sc2, zp2), 0.0)
        mx3 = jnp.maximum(_fq(mx_s, sc2, zp2), 0.0)
        sc3, zp3 = _qparams(mn3, mx3)
        o_ref[...] = _fq(dq2, sc3, zp3)


def _finish(a0, a1):
    return pl.pallas_call(
        _finish_body,
        grid=(2, _NBLK),
        in_specs=[
            pl.BlockSpec((_BLK, D), lambda p, i: (i, 0)),
            pl.BlockSpec((_BLK, D), lambda p, i: (i, 0)),
        ],
        out_specs=pl.BlockSpec((_BLK, D), lambda p, i: (i, 0)),
        out_shape=jax.ShapeDtypeStruct((N, D), jnp.float32),
        scratch_shapes=[
            pltpu.SMEM((1,), jnp.float32),
            pltpu.SMEM((1,), jnp.float32),
        ],
    )(a0, a1)


def kernel(x, edge_index):
    ei = edge_index.astype(jnp.int32)
    x = x.astype(jnp.float32)

    rmin, rmax = _rowminmax(x)
    omin, omax = _msg_minmax(rmin, rmax, ei)
    qx = _quant_x(x, omin, omax)
    zer = jnp.zeros((RT, D), jnp.float32)
    a0, a1 = _scatter_agg(qx, ei, zer)
    return _finish(a0, a1)
